# baseline (speedup 1.0000x reference)
import jax
import jax.numpy as jnp
from jax import lax
from jax.experimental import pallas as pl

NUM_CLASSES = 1000
ROWS_PER_BLOCK = 32


def _onehot_block(x_ref, o_ref):
    idx = x_ref[...]
    iota = lax.broadcasted_iota(jnp.int32, o_ref.shape, 2)
    o_ref[...] = (idx[:, :, None] == iota).astype(jnp.float32)


def kernel(x):
    n, m = x.shape
    grid = n // ROWS_PER_BLOCK
    return pl.pallas_call(
        _onehot_block,
        grid=(grid,),
        in_specs=[pl.BlockSpec((ROWS_PER_BLOCK, m), lambda i: (i, 0))],
        out_specs=pl.BlockSpec((ROWS_PER_BLOCK, m, NUM_CLASSES), lambda i: (i, 0, 0)),
        out_shape=jax.ShapeDtypeStruct((n, m, NUM_CLASSES), jnp.float32),
    )(x)
